# baseline (speedup 1.0000x reference)
import functools

import jax
import jax.numpy as jnp
from jax import lax
from jax.experimental import pallas as pl
from jax.experimental.pallas import tpu as pltpu
from jax.experimental.pallas import tpu_sc as plsc


def kernel(x, W):
    B, H = x.shape
    V, D = W.shape
    N = B * H

    info = plsc.get_sparse_core_info()
    NC, NS = info.num_cores, info.num_subcores
    NW = NC * NS
    rows_per_w = B // NW
    R = 8
    n_chunks = rows_per_w // R
    C = R * H

    mesh = plsc.VectorSubcoreMesh(core_axis_name="c", subcore_axis_name="s")

    @functools.partial(
        pl.kernel,
        mesh=mesh,
        compiler_params=pltpu.CompilerParams(use_tc_tiling_on_sc=False),
        out_type=jax.ShapeDtypeStruct((B, H, D), jnp.float32),
        scratch_types=[
            pltpu.VMEM((2, C), jnp.int32),
            pltpu.VMEM((2, C, D), jnp.float32),
            pltpu.SemaphoreType.DMA,
            pltpu.SemaphoreType.DMA,
        ],
    )
    def gather_kernel(table_hbm, idx_hbm, out_hbm, idx_v, rows_v, gsem, osem):
        wid = lax.axis_index("s") * NC + lax.axis_index("c")
        base = wid * rows_per_w

        def stage_and_gather(ci, slot):
            b0 = base + ci * R
            pltpu.sync_copy(idx_hbm.at[pl.ds(b0 * H, C)], idx_v.at[slot])
            pltpu.async_copy(
                table_hbm.at[idx_v.at[slot]], rows_v.at[slot], gsem
            )

        stage_and_gather(0, 0)

        def body(i, carry):
            slot = lax.rem(i, 2)
            nslot = lax.rem(i + 1, 2)
            b0 = base + i * R

            @pl.when(i >= 1)
            def _():
                pb0 = b0 - R
                for r in range(R):
                    pltpu.make_async_copy(
                        rows_v.at[nslot].at[pl.ds(r * H, H)],
                        out_hbm.at[pb0 + r],
                        osem,
                    ).wait()

            pltpu.make_async_copy(
                table_hbm.at[idx_v.at[slot]], rows_v.at[slot], gsem
            ).wait()

            @pl.when(i + 1 < n_chunks)
            def _():
                stage_and_gather(i + 1, nslot)

            for r in range(R):
                pltpu.async_copy(
                    rows_v.at[slot].at[pl.ds(r * H, H)],
                    out_hbm.at[b0 + r],
                    osem,
                )
            return carry

        lax.fori_loop(0, n_chunks, body, 0)

        lb0 = base + (n_chunks - 1) * R
        lslot = (n_chunks - 1) % 2
        for r in range(R):
            pltpu.make_async_copy(
                rows_v.at[lslot].at[pl.ds(r * H, H)],
                out_hbm.at[lb0 + r],
                osem,
            ).wait()

    return gather_kernel(W, x.reshape(N))


# speedup vs baseline: 1.0096x; 1.0096x over previous
import functools

import jax
import jax.numpy as jnp
from jax import lax
from jax.experimental import pallas as pl
from jax.experimental.pallas import tpu as pltpu
from jax.experimental.pallas import tpu_sc as plsc


def kernel(x, W):
    B, H = x.shape
    V, D = W.shape
    N = B * H

    info = plsc.get_sparse_core_info()
    NC, NS = info.num_cores, info.num_subcores
    NW = NC * NS
    rows_per_w = B // NW
    R = 16
    n_chunks = rows_per_w // R
    C = R * H

    mesh = plsc.VectorSubcoreMesh(core_axis_name="c", subcore_axis_name="s")

    @functools.partial(
        pl.kernel,
        mesh=mesh,
        compiler_params=pltpu.CompilerParams(
            use_tc_tiling_on_sc=False, needs_layout_passes=False
        ),
        out_type=jax.ShapeDtypeStruct((B, H, D), jnp.float32),
        scratch_types=[
            pltpu.VMEM((C,), jnp.int32),
            pltpu.VMEM((C, D), jnp.float32),
            pltpu.SemaphoreType.DMA,
            pltpu.SemaphoreType.DMA,
        ],
    )
    def gather_kernel(table_hbm, idx_hbm, out_hbm, idx_v, rows_v, gsem, osem):
        wid = lax.axis_index("s") * NC + lax.axis_index("c")
        base = wid * rows_per_w

        def body(i, carry):
            b0 = base + i * R
            pltpu.sync_copy(idx_hbm.at[pl.ds(b0 * H, C)], idx_v)
            pltpu.async_copy(table_hbm.at[idx_v], rows_v, gsem).wait()
            copies = [
                pltpu.async_copy(
                    rows_v.at[pl.ds(r * H, H)], out_hbm.at[b0 + r], osem
                )
                for r in range(R)
            ]
            for c in copies:
                c.wait()
            return carry

        lax.fori_loop(0, n_chunks, body, 0)

    return gather_kernel(W, x.reshape(N))
